# baseline (speedup 1.0000x reference)
import functools
import math

import jax
import jax.numpy as jnp
from jax import lax
from jax.experimental import pallas as pl
from jax.experimental.pallas import tpu as pltpu

_SMOOTHING = 0.1
_CONFIDENCE = 1.0 - _SMOOTHING
_IGNORE = 0
_ROWS = 64


def _body(pred_ref, tgt_ref, out_ref, acc_ref, *, batch, tlogt, eps):
    j = pl.program_id(0)
    nb = pl.num_programs(0)
    x = pred_ref[...].astype(jnp.float32)
    tgt = tgt_ref[...]
    validf = (tgt != _IGNORE).astype(jnp.float32)
    rowsum = jnp.sum(x, axis=1, keepdims=True)
    col = lax.broadcasted_iota(jnp.int32, x.shape, 1)
    gathered = jnp.sum(jnp.where(col == tgt, x, 0.0), axis=1, keepdims=True)
    part = jnp.sum(
        validf * (tlogt - eps * rowsum - (_CONFIDENCE - eps) * gathered)
    )

    @pl.when(j == 0)
    def _():
        acc_ref[0] = 0.0

    acc_ref[0] += part

    @pl.when(j == nb - 1)
    def _():
        out_ref[0, 0] = acc_ref[0] / batch


def kernel(pred_logprob, target):
    batch, vocab = pred_logprob.shape
    eps = _SMOOTHING / (vocab - 1)
    tlogt = (vocab - 1) * eps * math.log(eps) + _CONFIDENCE * math.log(
        _CONFIDENCE
    )
    pred_bf16 = pred_logprob.astype(jnp.bfloat16)
    tgt2 = target.reshape(batch, 1)
    out = pl.pallas_call(
        functools.partial(_body, batch=batch, tlogt=tlogt, eps=eps),
        grid=(batch // _ROWS,),
        in_specs=[
            pl.BlockSpec((_ROWS, vocab), lambda j: (j, 0)),
            pl.BlockSpec((_ROWS, 1), lambda j: (j, 0)),
        ],
        out_specs=pl.BlockSpec(
            (1, 1), lambda j: (0, 0), memory_space=pltpu.SMEM
        ),
        out_shape=jax.ShapeDtypeStruct((1, 1), jnp.float32),
        scratch_shapes=[pltpu.SMEM((1,), jnp.float32)],
        compiler_params=pltpu.CompilerParams(
            dimension_semantics=("arbitrary",)
        ),
    )(pred_bf16, tgt2)
    return out.reshape(())


# speedup vs baseline: 1.1198x; 1.1198x over previous
import functools
import math

import jax
import jax.numpy as jnp
from jax import lax
from jax.experimental import pallas as pl
from jax.experimental.pallas import tpu as pltpu

_SMOOTHING = 0.1
_CONFIDENCE = 1.0 - _SMOOTHING
_IGNORE = 0
_ROWS_PER_BLOCK = 64


def _body(pred_ref, tgt_ref, out_ref, acc_ref, *, batch, tlogt, eps):
    j = pl.program_id(0)
    nb = pl.num_programs(0)
    x = pred_ref[...]
    tgt = tgt_ref[...]
    validf = (tgt != _IGNORE).astype(jnp.float32)
    rowsum = jnp.sum(x, axis=1, keepdims=True)
    col = lax.broadcasted_iota(jnp.int32, x.shape, 1)
    gathered = jnp.sum(jnp.where(col == tgt, x, 0.0), axis=1, keepdims=True)
    part = jnp.sum(
        validf * (tlogt - eps * rowsum - (_CONFIDENCE - eps) * gathered)
    )

    @pl.when(j == 0)
    def _():
        acc_ref[0] = 0.0

    acc_ref[0] += part

    @pl.when(j == nb - 1)
    def _():
        out_ref[0, 0] = acc_ref[0] / batch


def kernel(pred_logprob, target):
    batch, vocab = pred_logprob.shape
    eps = _SMOOTHING / (vocab - 1)
    tlogt = (vocab - 1) * eps * math.log(eps) + _CONFIDENCE * math.log(
        _CONFIDENCE
    )
    rows = _ROWS_PER_BLOCK
    tgt2 = target.reshape(batch, 1)
    out = pl.pallas_call(
        functools.partial(_body, batch=batch, tlogt=tlogt, eps=eps),
        grid=(batch // rows,),
        in_specs=[
            pl.BlockSpec((rows, vocab), lambda j: (j, 0)),
            pl.BlockSpec((rows, 1), lambda j: (j, 0)),
        ],
        out_specs=pl.BlockSpec(
            (1, 1), lambda j: (0, 0), memory_space=pltpu.SMEM
        ),
        out_shape=jax.ShapeDtypeStruct((1, 1), jnp.float32),
        scratch_shapes=[pltpu.SMEM((1,), jnp.float32)],
        compiler_params=pltpu.CompilerParams(
            dimension_semantics=("arbitrary",)
        ),
    )(pred_logprob, tgt2)
    return out.reshape(())
